# Initial kernel scaffold; baseline (speedup 1.0000x reference)
import jax
import jax.numpy as jnp
from jax.experimental import pallas as pl


def kernel(token_ids, type_ids, token_table, type_table, pos_table, ln_gamma, ln_beta):
    raise NotImplementedError("write your pallas kernel here")


# speedup vs baseline: 2.0364x; 2.0364x over previous
import functools

import jax
import jax.numpy as jnp
from jax import lax
from jax.experimental import pallas as pl
from jax.experimental.pallas import tpu as pltpu
from jax.experimental.pallas import tpu_sc as plsc

HIDDEN = 64
NC, NS = 2, 16
NW = NC * NS
GSZ = 128


def _sc_gather(table, idx2d, rows):
    ng_total = idx2d.shape[0]
    ng = ng_total // NW
    rpw = ng * GSZ

    mesh = plsc.VectorSubcoreMesh(core_axis_name="c", subcore_axis_name="s")

    @functools.partial(
        pl.kernel,
        mesh=mesh,
        compiler_params=pltpu.CompilerParams(use_tc_tiling_on_sc=False),
        out_type=jax.ShapeDtypeStruct((rows, HIDDEN), jnp.float32),
        scratch_types=[
            pltpu.VMEM((ng, GSZ), jnp.int32),
            pltpu.VMEM((2, GSZ, HIDDEN), jnp.float32),
            pltpu.SemaphoreType.DMA,
        ],
    )
    def k(table_hbm, idx_hbm, out_hbm, idx_v, rows_v, sem):
        wid = lax.axis_index("s") * NC + lax.axis_index("c")
        pltpu.sync_copy(idx_hbm.at[pl.ds(wid * ng, ng)], idx_v)

        def step(j, carry):
            pltpu.async_copy(table_hbm.at[idx_v.at[j]], rows_v.at[0], sem).wait()
            pltpu.sync_copy(
                rows_v.at[0], out_hbm.at[pl.ds(wid * rpw + j * GSZ, GSZ)]
            )
            return carry

        lax.fori_loop(0, ng, step, 0)

    return k(table, idx2d)


def _tc_epilogue(tok, type_ids, type_table, pos, gamma, beta):
    B, S = type_ids.shape
    BB = 64
    ntypes = type_table.shape[0]

    def body(tok_ref, tid_ref, ttab_ref, pos_ref, g_ref, b_ref, o_ref):
        t = tid_ref[...]
        acc = tok_ref[...] + pos_ref[...][None, :, :]
        ttab = ttab_ref[...]
        tb = lax.broadcast_in_dim(t, acc.shape, (0, 1))
        for k in range(ntypes):
            acc += jnp.where(tb == k, ttab[k][None, None, :], 0.0)
        mean = jnp.mean(acc, axis=-1, keepdims=True)
        c = acc - mean
        var = jnp.mean(c * c, axis=-1, keepdims=True)
        o_ref[...] = (
            c * lax.rsqrt(var + 1e-5) * g_ref[...][0][None, None, :]
            + b_ref[...][0][None, None, :]
        )

    return pl.pallas_call(
        body,
        grid=(B // BB,),
        in_specs=[
            pl.BlockSpec((BB, S, HIDDEN), lambda i: (i, 0, 0)),
            pl.BlockSpec((BB, S), lambda i: (i, 0)),
            pl.BlockSpec((ntypes, HIDDEN), lambda i: (0, 0)),
            pl.BlockSpec((S, HIDDEN), lambda i: (0, 0)),
            pl.BlockSpec((1, HIDDEN), lambda i: (0, 0)),
            pl.BlockSpec((1, HIDDEN), lambda i: (0, 0)),
        ],
        out_specs=pl.BlockSpec((BB, S, HIDDEN), lambda i: (i, 0, 0)),
        out_shape=jax.ShapeDtypeStruct((B, S, HIDDEN), jnp.float32),
    )(tok, type_ids, type_table, pos, gamma.reshape(1, -1), beta.reshape(1, -1))


def kernel(token_ids, type_ids, token_table, type_table, pos_table, ln_gamma, ln_beta):
    B, S = token_ids.shape
    rows = B * S
    idx2d = token_ids.reshape(rows // GSZ, GSZ).astype(jnp.int32)
    tok = _sc_gather(token_table, idx2d, rows)
    return _tc_epilogue(
        tok.reshape(B, S, HIDDEN),
        type_ids.astype(jnp.int32),
        type_table,
        pos_table[:S],
        ln_gamma,
        ln_beta,
    )


# speedup vs baseline: 2.2096x; 1.0851x over previous
import functools

import jax
import jax.numpy as jnp
from jax import lax
from jax.experimental import pallas as pl
from jax.experimental.pallas import tpu as pltpu
from jax.experimental.pallas import tpu_sc as plsc

HIDDEN = 64
NC, NS = 2, 16
NW = NC * NS
GSZ = 128


def _sc_gather(table, idx2d, rows):
    ng_total = idx2d.shape[0]
    ng = ng_total // NW
    rpw = ng * GSZ
    NBUF = 8
    LOOK = 4

    mesh = plsc.VectorSubcoreMesh(core_axis_name="c", subcore_axis_name="s")

    @functools.partial(
        pl.kernel,
        mesh=mesh,
        compiler_params=pltpu.CompilerParams(use_tc_tiling_on_sc=False),
        out_type=jax.ShapeDtypeStruct((rows, HIDDEN), jnp.float32),
        scratch_types=[
            pltpu.VMEM((ng, GSZ), jnp.int32),
            pltpu.VMEM((NBUF, GSZ, HIDDEN), jnp.float32),
            pltpu.SemaphoreType.DMA((NBUF,)),
            pltpu.SemaphoreType.DMA((NBUF,)),
        ],
    )
    def k(table_hbm, idx_hbm, out_hbm, idx_v, rows_v, gsem, osem):
        wid = lax.axis_index("s") * NC + lax.axis_index("c")
        pltpu.sync_copy(idx_hbm.at[pl.ds(wid * ng, ng)], idx_v)

        def fire_gather(j, b):
            pltpu.async_copy(table_hbm.at[idx_v.at[j]], rows_v.at[b], gsem.at[b])

        def wait_gather(b):
            pltpu.make_async_copy(
                out_hbm.at[pl.ds(0, GSZ)], rows_v.at[b], gsem.at[b]
            ).wait()

        def wait_outcopy(b):
            pltpu.make_async_copy(
                rows_v.at[b], out_hbm.at[pl.ds(0, GSZ)], osem.at[b]
            ).wait()

        for j0 in range(LOOK):
            fire_gather(j0, j0)

        def body(j, carry):
            b = lax.rem(j, NBUF)
            wait_gather(b)
            pltpu.async_copy(
                rows_v.at[b],
                out_hbm.at[pl.ds(wid * rpw + j * GSZ, GSZ)],
                osem.at[b],
            )
            jn = j + LOOK
            bn = lax.rem(jn, NBUF)

            @pl.when(jn < ng)
            def _():
                @pl.when(j >= NBUF - LOOK)
                def _():
                    wait_outcopy(bn)

                fire_gather(jn, bn)

            return carry

        lax.fori_loop(0, ng, body, 0)
        for b in range(NBUF):
            wait_outcopy(b)

    return k(table, idx2d)


def _tc_epilogue(tok, type_ids, type_table, pos, gamma, beta):
    B, S = type_ids.shape
    BB = 64
    ntypes = type_table.shape[0]

    def body(tok_ref, tid_ref, ttab_ref, pos_ref, g_ref, b_ref, o_ref):
        t = tid_ref[...]
        acc = tok_ref[...] + pos_ref[...][None, :, :]
        ttab = ttab_ref[...]
        tb = lax.broadcast_in_dim(t, acc.shape, (0, 1))
        for k in range(ntypes):
            acc += jnp.where(tb == k, ttab[k][None, None, :], 0.0)
        mean = jnp.mean(acc, axis=-1, keepdims=True)
        c = acc - mean
        var = jnp.mean(c * c, axis=-1, keepdims=True)
        o_ref[...] = (
            c * lax.rsqrt(var + 1e-5) * g_ref[...][0][None, None, :]
            + b_ref[...][0][None, None, :]
        )

    return pl.pallas_call(
        body,
        grid=(B // BB,),
        in_specs=[
            pl.BlockSpec((BB, S, HIDDEN), lambda i: (i, 0, 0)),
            pl.BlockSpec((BB, S), lambda i: (i, 0)),
            pl.BlockSpec((ntypes, HIDDEN), lambda i: (0, 0)),
            pl.BlockSpec((S, HIDDEN), lambda i: (0, 0)),
            pl.BlockSpec((1, HIDDEN), lambda i: (0, 0)),
            pl.BlockSpec((1, HIDDEN), lambda i: (0, 0)),
        ],
        out_specs=pl.BlockSpec((BB, S, HIDDEN), lambda i: (i, 0, 0)),
        out_shape=jax.ShapeDtypeStruct((B, S, HIDDEN), jnp.float32),
    )(tok, type_ids, type_table, pos, gamma.reshape(1, -1), beta.reshape(1, -1))


def kernel(token_ids, type_ids, token_table, type_table, pos_table, ln_gamma, ln_beta):
    B, S = token_ids.shape
    rows = B * S
    idx2d = token_ids.reshape(rows // GSZ, GSZ).astype(jnp.int32)
    tok = _sc_gather(token_table, idx2d, rows)
    return _tc_epilogue(
        tok.reshape(B, S, HIDDEN),
        type_ids.astype(jnp.int32),
        type_table,
        pos_table[:S],
        ln_gamma,
        ln_beta,
    )
